# baseline (speedup 1.0000x reference)
import functools

import jax
import jax.numpy as jnp
from jax import lax
from jax.experimental import pallas as pl
from jax.experimental.pallas import tpu as pltpu
from jax.experimental.pallas import tpu_sc as plsc

_NUM_CORES = 2
_NUM_SUBCORES = 16
_NW = _NUM_CORES * _NUM_SUBCORES
_CHUNK = 128


def _make_gather(n_chunks: int, d: int):
    mesh = plsc.VectorSubcoreMesh(core_axis_name="c", subcore_axis_name="s")

    @functools.partial(
        pl.kernel,
        out_type=jax.ShapeDtypeStruct((_NW, n_chunks, _CHUNK, d), jnp.float32),
        mesh=mesh,
        scratch_types=[
            pltpu.VMEM((n_chunks, _CHUNK), jnp.int32),
            pltpu.VMEM((_CHUNK, d), jnp.float32),
            pltpu.SemaphoreType.DMA,
        ],
        compiler_params=pltpu.CompilerParams(use_tc_tiling_on_sc=False),
    )
    def gather(table_hbm, idx_hbm, out_hbm, idx_v, rows_v, sem):
        wid = lax.axis_index("s") * _NUM_CORES + lax.axis_index("c")
        pltpu.sync_copy(idx_hbm.at[wid], idx_v)

        def step(j, carry):
            pltpu.async_copy(table_hbm.at[idx_v.at[j]], rows_v, sem).wait()
            pltpu.sync_copy(rows_v, out_hbm.at[wid, j])
            return carry

        lax.fori_loop(0, n_chunks, step, 0)

    return gather


def kernel(token_ids, embed):
    shape = token_ids.shape
    d = embed.shape[1]
    flat = token_ids.reshape(-1).astype(jnp.int32)
    n = flat.shape[0]
    per_w = -(-n // (_NW * _CHUNK)) * _CHUNK
    total = per_w * _NW
    if total != n:
        flat = jnp.concatenate([flat, jnp.zeros((total - n,), jnp.int32)])
    n_chunks = per_w // _CHUNK
    idx = flat.reshape(_NW, n_chunks, _CHUNK)
    out = _make_gather(n_chunks, d)(embed, idx)
    out = out.reshape(total, d)[:n]
    return out.reshape(*shape, d)


# speedup vs baseline: 1.0800x; 1.0800x over previous
import functools

import jax
import jax.numpy as jnp
from jax import lax
from jax.experimental import pallas as pl
from jax.experimental.pallas import tpu as pltpu
from jax.experimental.pallas import tpu_sc as plsc

_NUM_CORES = 2
_NUM_SUBCORES = 16
_NW = _NUM_CORES * _NUM_SUBCORES
_CHUNK = 128
_RING = 8
_AHEAD = 4


def _make_gather(n_chunks: int, d: int):
    mesh = plsc.VectorSubcoreMesh(core_axis_name="c", subcore_axis_name="s")
    n = n_chunks
    r, a = _RING, _AHEAD
    if n <= r:
        r = a = 0

    @functools.partial(
        pl.kernel,
        out_type=jax.ShapeDtypeStruct((_NW, n, _CHUNK, d), jnp.float32),
        mesh=mesh,
        scratch_types=[
            pltpu.VMEM((n, _CHUNK), jnp.int32),
            pltpu.VMEM((max(r, 1), _CHUNK, d), jnp.float32),
        ] + [pltpu.SemaphoreType.DMA] * max(r, 1),
        compiler_params=pltpu.CompilerParams(use_tc_tiling_on_sc=False),
    )
    def gather(table_hbm, idx_hbm, out_hbm, idx_v, rows_v, *sems):
        wid = lax.axis_index("s") * _NUM_CORES + lax.axis_index("c")
        pltpu.sync_copy(idx_hbm.at[wid], idx_v)

        if r == 0:
            def step(j, carry):
                pltpu.async_copy(table_hbm.at[idx_v.at[j]], rows_v.at[0],
                                 sems[0]).wait()
                pltpu.sync_copy(rows_v.at[0], out_hbm.at[wid, j])
                return carry
            lax.fori_loop(0, n, step, 0)
            return

        def start_gather(j, slot):
            pltpu.async_copy(table_hbm.at[idx_v.at[j]], rows_v.at[slot],
                             sems[slot])

        def wait_slot(slot):
            pltpu.make_async_copy(table_hbm.at[idx_v.at[0]], rows_v.at[slot],
                                  sems[slot]).wait()

        def start_out(j, slot):
            pltpu.async_copy(rows_v.at[slot], out_hbm.at[wid, j], sems[slot])

        def full_step(j, slot, slot_a):
            wait_slot(slot_a)
            start_gather(j + a, slot_a)
            wait_slot(slot)
            start_out(j, slot)

        for j in range(a):
            start_gather(j, j % r)
        for j in range(r - a):
            start_gather(j + a, (j + a) % r)
            wait_slot(j % r)
            start_out(j, j % r)

        main_iters = (n - r) // r
        rem = (n - r) % r

        def main_body(g, carry):
            j0 = (r - a) + g * r
            for b in range(r):
                full_step(j0 + b, (r - a + b) % r, (b + (r - a) + a) % r)
            return carry
        lax.fori_loop(0, main_iters, main_body, 0)

        for i in range(rem):
            j = (r - a) + main_iters * r + i
            full_step(j, j % r, (j + a) % r)
        for j in range(n - a, n):
            wait_slot(j % r)
            start_out(j, j % r)
        for b in range(r):
            wait_slot(b)

    return gather


def kernel(token_ids, embed):
    shape = token_ids.shape
    d = embed.shape[1]
    flat = token_ids.reshape(-1).astype(jnp.int32)
    n = flat.shape[0]
    per_w = -(-n // (_NW * _CHUNK)) * _CHUNK
    total = per_w * _NW
    if total != n:
        flat = jnp.concatenate([flat, jnp.zeros((total - n,), jnp.int32)])
    n_chunks = per_w // _CHUNK
    idx = flat.reshape(_NW, n_chunks, _CHUNK)
    out = _make_gather(n_chunks, d)(embed, idx)
    out = out.reshape(total, d)[:n]
    return out.reshape(*shape, d)
